# Initial kernel scaffold; baseline (speedup 1.0000x reference)
import jax
import jax.numpy as jnp
from jax.experimental import pallas as pl


def kernel(x, edge_index, lin_weight, att_src, att_dst, bias):
    raise NotImplementedError("write your pallas kernel here")


# speedup vs baseline: 1.0370x; 1.0370x over previous
import functools

import jax
import jax.numpy as jnp
import numpy as np
from jax.experimental import pallas as pl
from jax.experimental.pallas import tpu as pltpu

N = 50000
E = 800000
IN_CH = 16
HEADS = 2
OUT_CH = 32
NEG_SLOPE = 0.2

ROW_BLK = 1024
NT = 50176
EP = 851968
TRASH = N + 64


def _dense_body(x_ref, wt_ref, asc_ref, adc_ref, h_ref, a_ref, b_ref, mx_ref):
    i = pl.program_id(0)
    xb = x_ref[...]
    h = jax.lax.dot_general(xb, wt_ref[...], (((1,), (0,)), ((), ())),
                            preferred_element_type=jnp.float32)
    h_ref[...] = h
    hh = h.reshape(ROW_BLK, HEADS, OUT_CH)
    asrc = (hh * asc_ref[...]).sum(-1)
    adst = (hh * adc_ref[...]).sum(-1)
    a_ref[...] = asrc
    b_ref[...] = adst
    blk_mx = jnp.max(asrc, axis=0) + jnp.max(adst, axis=0)
    blk_mx = jnp.broadcast_to(blk_mx[None, :], (8, HEADS))

    @pl.when(i == 0)
    def _():
        mx_ref[...] = blk_mx

    @pl.when(i > 0)
    def _():
        mx_ref[...] = jnp.maximum(mx_ref[...], blk_mx)


def _dense_stage(xp, lin_weight, att_src, att_dst):
    wt = lin_weight.T
    grid = NT // ROW_BLK
    h, asrc, adst, gb = pl.pallas_call(
        _dense_body,
        grid=(grid,),
        in_specs=[
            pl.BlockSpec((ROW_BLK, IN_CH), lambda i: (i, 0)),
            pl.BlockSpec((IN_CH, HEADS * OUT_CH), lambda i: (0, 0)),
            pl.BlockSpec((1, HEADS, OUT_CH), lambda i: (0, 0, 0)),
            pl.BlockSpec((1, HEADS, OUT_CH), lambda i: (0, 0, 0)),
        ],
        out_specs=[
            pl.BlockSpec((ROW_BLK, HEADS * OUT_CH), lambda i: (i, 0)),
            pl.BlockSpec((ROW_BLK, HEADS), lambda i: (i, 0)),
            pl.BlockSpec((ROW_BLK, HEADS), lambda i: (i, 0)),
            pl.BlockSpec((8, HEADS), lambda i: (0, 0)),
        ],
        out_shape=[
            jax.ShapeDtypeStruct((NT, HEADS * OUT_CH), jnp.float32),
            jax.ShapeDtypeStruct((NT, HEADS), jnp.float32),
            jax.ShapeDtypeStruct((NT, HEADS), jnp.float32),
            jax.ShapeDtypeStruct((8, HEADS), jnp.float32),
        ],
    )(xp, wt, att_src, att_dst)
    return h, asrc, adst, gb[0]


def kernel(x, edge_index, lin_weight, att_src, att_dst, bias):
    xp = jnp.pad(x, ((0, NT - N), (0, 0)))
    h, asrc, adst, gb = _dense_stage(xp, lin_weight, att_src, att_dst)

    loop = jnp.arange(N, dtype=edge_index.dtype)
    src = jnp.concatenate([edge_index[0], loop,
                           jnp.full((EP - E - N,), TRASH, edge_index.dtype)])
    dst = jnp.concatenate([edge_index[1], loop,
                           jnp.full((EP - E - N,), TRASH, edge_index.dtype)])

    alpha = asrc[src] + adst[dst]
    alpha = jnp.where(alpha >= 0, alpha, NEG_SLOPE * alpha)
    ex = jnp.exp(alpha - gb[None, :])
    denom = jax.ops.segment_sum(ex, dst, num_segments=NT)
    coef = ex / (denom[dst] + 1e-16)
    msg = coef[:, :, None] * h[src].reshape(EP, HEADS, OUT_CH)
    out = jax.ops.segment_sum(msg, dst, num_segments=NT)
    out = out[:N].reshape(N, HEADS * OUT_CH) + bias
    return out


# speedup vs baseline: 26.5771x; 25.6296x over previous
import functools

import jax
import jax.numpy as jnp
from jax import lax
from jax.experimental import pallas as pl
from jax.experimental.pallas import tpu as pltpu
from jax.experimental.pallas import tpu_sc as plsc

N = 50000
E = 800000
IN_CH = 16
HEADS = 2
OUT_CH = 32
NEG_SLOPE = 0.2

ROW_BLK = 1024
NT = 50176
TRASH = N + 64
NSUB = 16
C = 320
NK = 167
TPW = C * NK
EP = NSUB * TPW
CA = 160
NKA = 334
D = 160
ND = TPW // D
NPT = NT // NSUB
NZD = 19
NZ = NPT - NZD * D
HN = HEADS * NT
QN = HEADS * EP

_MESH = dict(core_axis_name="c", subcore_axis_name="s")
_CP = pltpu.CompilerParams(needs_layout_passes=False)


def _dense_body(x_ref, wt_ref, asc_ref, adc_ref, h_ref, a_ref, b_ref, mx_ref):
    i = pl.program_id(0)
    xb = x_ref[...]
    h = lax.dot_general(xb, wt_ref[...], (((1,), (0,)), ((), ())),
                        preferred_element_type=jnp.float32)
    h_ref[...] = h
    hh = h.reshape(ROW_BLK, HEADS, OUT_CH)
    asrc = (hh * asc_ref[...]).sum(-1)
    adst = (hh * adc_ref[...]).sum(-1)
    a_ref[...] = asrc
    b_ref[...] = adst
    blk_mx = jnp.max(asrc, axis=0) + jnp.max(adst, axis=0)
    blk_mx = jnp.broadcast_to(blk_mx[None, :], (8, HEADS))

    @pl.when(i == 0)
    def _():
        mx_ref[...] = blk_mx

    @pl.when(i > 0)
    def _():
        mx_ref[...] = jnp.maximum(mx_ref[...], blk_mx)


def _dense_stage(xp, lin_weight, att_src, att_dst):
    wt = lin_weight.T
    grid = NT // ROW_BLK
    h, asrc, adst, gb = pl.pallas_call(
        _dense_body,
        grid=(grid,),
        in_specs=[
            pl.BlockSpec((ROW_BLK, IN_CH), lambda i: (i, 0)),
            pl.BlockSpec((IN_CH, HEADS * OUT_CH), lambda i: (0, 0)),
            pl.BlockSpec((1, HEADS, OUT_CH), lambda i: (0, 0, 0)),
            pl.BlockSpec((1, HEADS, OUT_CH), lambda i: (0, 0, 0)),
        ],
        out_specs=[
            pl.BlockSpec((ROW_BLK, HEADS * OUT_CH), lambda i: (i, 0)),
            pl.BlockSpec((ROW_BLK, HEADS), lambda i: (i, 0)),
            pl.BlockSpec((ROW_BLK, HEADS), lambda i: (i, 0)),
            pl.BlockSpec((8, HEADS), lambda i: (0, 0)),
        ],
        out_shape=[
            jax.ShapeDtypeStruct((NT, HEADS * OUT_CH), jnp.float32),
            jax.ShapeDtypeStruct((NT, HEADS), jnp.float32),
            jax.ShapeDtypeStruct((NT, HEADS), jnp.float32),
            jax.ShapeDtypeStruct((8, HEADS), jnp.float32),
        ],
    )(xp, wt, att_src, att_dst)
    return h, asrc, adst, gb[0]


def _sc_a1_body(src_h, asrc_h, axr_h, tab, srcb, exb):
    c = lax.axis_index("c")
    s = lax.axis_index("s")
    pltpu.sync_copy(asrc_h.at[pl.ds(c * NT, NT)], tab)

    def _chunk(k, _):
        base = s * TPW + k * C
        pltpu.sync_copy(src_h.at[pl.ds(base, C)], srcb)

        def _grp(v, _):
            si = srcb[pl.ds(v * 16, 16)]
            exb[pl.ds(v * 16, 16)] = plsc.load_gather(tab, [si])
            return 0
        lax.fori_loop(0, C // 16, _grp, 0)
        pltpu.sync_copy(exb, axr_h.at[pl.ds(c * EP + base, C)])
        return 0
    lax.fori_loop(0, NK, _chunk, 0)


_sc_a1 = functools.partial(
    pl.kernel,
    _sc_a1_body,
    out_type=jax.ShapeDtypeStruct((QN,), jnp.float32),
    mesh=plsc.VectorSubcoreMesh(**_MESH),
    compiler_params=_CP,
    scratch_types=[
        pltpu.VMEM((NT,), jnp.float32),
        pltpu.VMEM((C,), jnp.int32),
        pltpu.VMEM((C,), jnp.float32),
    ],
)()


def _sc_a2_body(dst_h, axr_h, adst_h, gb_h, ex_h, den_h,
                denom_sp, tab, dstb, axb, exb, gbuf):
    c = lax.axis_index("c")
    s = lax.axis_index("s")
    zf = jnp.zeros((16,), jnp.float32)

    pltpu.sync_copy(gb_h, gbuf)
    gv = jnp.where(c == 0, gbuf[0], gbuf[1])
    pltpu.sync_copy(adst_h.at[pl.ds(c * NT, NT)], tab)

    def _zero(v, _):
        exb[pl.ds(v * 16, 16)] = zf
        return 0
    lax.fori_loop(0, C // 16, _zero, 0)
    for j in range(NPT // C):
        pltpu.sync_copy(exb, denom_sp.at[pl.ds(s * NPT + j * C, C)])
    _rm = NPT - (NPT // C) * C
    if _rm:
        pltpu.sync_copy(exb.at[pl.ds(0, _rm)],
                        denom_sp.at[pl.ds(s * NPT + (NPT // C) * C, _rm)])
    plsc.subcore_barrier()

    def _chunk(k, _):
        base = s * TPW + k * C
        pltpu.sync_copy(dst_h.at[pl.ds(base, C)], dstb)
        pltpu.sync_copy(axr_h.at[pl.ds(c * EP + base, C)], axb)

        def _grp(v, _):
            di = dstb[pl.ds(v * 16, 16)]
            al = axb[pl.ds(v * 16, 16)] + plsc.load_gather(tab, [di])
            al = jnp.where(al >= 0, al, al * NEG_SLOPE)
            exb[pl.ds(v * 16, 16)] = jnp.exp(al - gv)
            return 0
        lax.fori_loop(0, C // 16, _grp, 0)
        pltpu.sync_copy(exb, ex_h.at[pl.ds(c * EP + base, C)])
        pltpu.sync_copy(exb, denom_sp.at[dstb], add=True)
        return 0
    lax.fori_loop(0, NK, _chunk, 0)
    plsc.subcore_barrier()
    for j in range(NPT // C):
        pltpu.sync_copy(denom_sp.at[pl.ds(s * NPT + j * C, C)], exb)
        pltpu.sync_copy(exb, den_h.at[pl.ds(c * NT + s * NPT + j * C, C)])
    if _rm:
        _o = s * NPT + (NPT // C) * C
        pltpu.sync_copy(denom_sp.at[pl.ds(_o, _rm)], exb.at[pl.ds(0, _rm)])
        pltpu.sync_copy(exb.at[pl.ds(0, _rm)],
                        den_h.at[pl.ds(c * NT + _o, _rm)])


_sc_a2 = functools.partial(
    pl.kernel,
    _sc_a2_body,
    out_type=(jax.ShapeDtypeStruct((QN,), jnp.float32),
              jax.ShapeDtypeStruct((HN,), jnp.float32)),
    mesh=plsc.VectorSubcoreMesh(**_MESH),
    compiler_params=_CP,
    scratch_types=[
        pltpu.VMEM_SHARED((NT,), jnp.float32),
        pltpu.VMEM((NT,), jnp.float32),
        pltpu.VMEM((C,), jnp.int32),
        pltpu.VMEM((C,), jnp.float32),
        pltpu.VMEM((C,), jnp.float32),
        pltpu.VMEM((HEADS, 16), jnp.float32),
    ],
)()


def _sc_a3_body(src_h, dst_h, ex_h, den_h, h_h, msg_h,
                tab, srcb, dstb, exb, coefb, h_rows, msg, msg2, sem):
    c = lax.axis_index("c")
    s = lax.axis_index("s")

    pltpu.sync_copy(den_h.at[pl.ds(c * NT, NT)], tab)

    def _recip(v, _):
        d = tab[pl.ds(v * 16, 16)]
        tab[pl.ds(v * 16, 16)] = 1.0 / (d + 1e-16)
        return 0
    lax.fori_loop(0, NT // 16, _recip, 0)

    def _chunk(k, _):
        base = s * TPW + k * CA
        pltpu.sync_copy(src_h.at[pl.ds(base, CA)], srcb)
        pltpu.sync_copy(dst_h.at[pl.ds(base, CA)], dstb)
        pltpu.sync_copy(ex_h.at[pl.ds(c * EP + base, CA)], exb)
        pltpu.async_copy(h_h.at[srcb], h_rows, sem).wait()

        def _coef(v, _):
            di = dstb[pl.ds(v * 16, 16)]
            coefb[pl.ds(v * 16, 16)] = (exb[pl.ds(v * 16, 16)]
                                        * plsc.load_gather(tab, [di]))
            return 0
        lax.fori_loop(0, CA // 16, _coef, 0)

        def _msg(g, _):
            cvec = coefb[pl.ds(g * 16, 16)]
            base_e = g * 16
            for j in range(16):
                e = base_e + j
                lo = jnp.where(c == 0, h_rows[e, pl.ds(0, 16)],
                               h_rows[e, pl.ds(32, 16)])
                hi = jnp.where(c == 0, h_rows[e, pl.ds(16, 16)],
                               h_rows[e, pl.ds(48, 16)])
                msg[pl.ds(e * 16, 16)] = lo * cvec[j]
                msg2[pl.ds(e * 16, 16)] = hi * cvec[j]
            return 0
        lax.fori_loop(0, CA // 16, _msg, 0)
        pltpu.sync_copy(msg, msg_h.at[pl.ds((c * EP + base) * 16, CA * 16)])
        pltpu.sync_copy(msg2,
                        msg_h.at[pl.ds((QN + c * EP + base) * 16, CA * 16)])
        return 0
    lax.fori_loop(0, NKA, _chunk, 0)


_sc_a3 = functools.partial(
    pl.kernel,
    _sc_a3_body,
    out_type=jax.ShapeDtypeStruct((2 * QN * 16,), jnp.float32),
    mesh=plsc.VectorSubcoreMesh(**_MESH),
    compiler_params=_CP,
    scratch_types=[
        pltpu.VMEM((NT,), jnp.float32),
        pltpu.VMEM((CA,), jnp.int32),
        pltpu.VMEM((CA,), jnp.int32),
        pltpu.VMEM((CA,), jnp.float32),
        pltpu.VMEM((CA,), jnp.float32),
        pltpu.VMEM((CA, 128), jnp.float32),
        pltpu.VMEM((CA * 16,), jnp.float32),
        pltpu.VMEM((CA * 16,), jnp.float32),
        pltpu.SemaphoreType.DMA,
    ],
)()


def _sc_b_body(dst_h, msg_h, out_h, out_sp, dstd, idxf, msgf):
    c = lax.axis_index("c")
    s = lax.axis_index("s")
    zf = jnp.zeros((16,), jnp.float32)
    lanes = lax.iota(jnp.int32, 16)
    DW = D * 16
    NPW = NPT * 16
    NFD = NPW // DW
    NFR = NPW - NFD * DW

    def _zero(v, _):
        msgf[pl.ds(v * 16, 16)] = zf
        return 0

    def _zero_outsp():
        for j in range(NFD):
            pltpu.sync_copy(msgf, out_sp.at[pl.ds(s * NPW + j * DW, DW)])
        if NFR:
            pltpu.sync_copy(msgf.at[pl.ds(0, NFR)],
                            out_sp.at[pl.ds(s * NPW + NFD * DW, NFR)])

    def _flush(off):
        for j in range(NFD):
            pltpu.sync_copy(out_sp.at[pl.ds(s * NPW + j * DW, DW)], msgf)
            pltpu.sync_copy(msgf, out_h.at[pl.ds(off + j * DW, DW)])
        if NFR:
            pltpu.sync_copy(out_sp.at[pl.ds(s * NPW + NFD * DW, NFR)],
                            msgf.at[pl.ds(0, NFR)])
            pltpu.sync_copy(msgf.at[pl.ds(0, NFR)],
                            out_h.at[pl.ds(off + NFD * DW, NFR)])

    lax.fori_loop(0, DW // 16, _zero, 0)
    _zero_outsp()
    plsc.subcore_barrier()

    for f in range(2):
        def _drain(kk, _):
            base = s * TPW + kk * D
            pltpu.sync_copy(dst_h.at[pl.ds(base, D)], dstd)
            pltpu.sync_copy(msg_h.at[pl.ds((f * QN + c * EP + base) * 16,
                                           DW)], msgf)

            def _idx(v, _):
                dv = dstd[pl.ds(v * 16, 16)] * 16
                for j in range(16):
                    idxf[pl.ds((v * 16 + j) * 16, 16)] = dv[j] + lanes
                return 0
            lax.fori_loop(0, D // 16, _idx, 0)
            pltpu.sync_copy(msgf, out_sp.at[idxf], add=True)
            return 0
        lax.fori_loop(0, ND, _drain, 0)
        plsc.subcore_barrier()
        _flush((f * HN + c * NT + s * NPT) * 16)
        if f == 0:
            lax.fori_loop(0, DW // 16, _zero, 0)
            _zero_outsp()
            plsc.subcore_barrier()


_sc_b = functools.partial(
    pl.kernel,
    _sc_b_body,
    out_type=jax.ShapeDtypeStruct((2 * HN * 16,), jnp.float32),
    mesh=plsc.VectorSubcoreMesh(**_MESH),
    compiler_params=_CP,
    scratch_types=[
        pltpu.VMEM_SHARED((NT * 16,), jnp.float32),
        pltpu.VMEM((D,), jnp.int32),
        pltpu.VMEM((D * 16,), jnp.int32),
        pltpu.VMEM((D * 16,), jnp.float32),
    ],
)()


def kernel(x, edge_index, lin_weight, att_src, att_dst, bias):
    xp = jnp.pad(x, ((0, NT - N), (0, 0)))
    h, asrc, adst, gb = _dense_stage(xp, lin_weight, att_src, att_dst)

    h128 = jnp.pad(h, ((0, 0), (0, 128 - HEADS * OUT_CH)))
    asrc2 = asrc.T.reshape(HN)
    adst2 = adst.T.reshape(HN)
    gb2 = jnp.broadcast_to(gb[:, None], (HEADS, 16))

    loop = jnp.arange(N, dtype=jnp.int32)
    pad = jnp.full((EP - E - N,), TRASH, jnp.int32)
    src = jnp.concatenate([edge_index[0].astype(jnp.int32), loop, pad])
    dst = jnp.concatenate([edge_index[1].astype(jnp.int32), loop, pad])

    axr = _sc_a1(src, asrc2)
    ex, den = _sc_a2(dst, axr, adst2, gb2)
    msgq = _sc_a3(src, dst, ex, den, h128)
    oh = _sc_b(dst, msgq).reshape(2 * HN, 16)
    out = jnp.concatenate([oh[:N], oh[HN:HN + N],
                           oh[NT:NT + N], oh[HN + NT:HN + NT + N]],
                          axis=1) + bias
    return out


# speedup vs baseline: 28.6522x; 1.0781x over previous
import functools

import jax
import jax.numpy as jnp
from jax import lax
from jax.experimental import pallas as pl
from jax.experimental.pallas import tpu as pltpu
from jax.experimental.pallas import tpu_sc as plsc

N = 50000
E = 800000
IN_CH = 16
HEADS = 2
OUT_CH = 32
NEG_SLOPE = 0.2

ROW_BLK = 1024
NT = 50176
TRASH = N + 64
NSUB = 16
C = 320
NK = 167
TPW = C * NK
EP = NSUB * TPW
CA = 160
NKA = 334
D = 160
ND = TPW // D
NPT = NT // NSUB
NZD = 19
NZ = NPT - NZD * D
HN = HEADS * NT
QN = HEADS * EP

_MESH = dict(core_axis_name="c", subcore_axis_name="s")
_CP = pltpu.CompilerParams(needs_layout_passes=False)


def _dense_body(x_ref, wt_ref, asc_ref, adc_ref, h_ref, a_ref, b_ref, mx_ref):
    i = pl.program_id(0)
    xb = x_ref[...]
    h = lax.dot_general(xb, wt_ref[...], (((1,), (0,)), ((), ())),
                        preferred_element_type=jnp.float32)
    h_ref[...] = h
    hh = h.reshape(ROW_BLK, HEADS, OUT_CH)
    asrc = (hh * asc_ref[...]).sum(-1)
    adst = (hh * adc_ref[...]).sum(-1)
    a_ref[...] = asrc
    b_ref[...] = adst
    blk_mx = jnp.max(asrc, axis=0) + jnp.max(adst, axis=0)
    blk_mx = jnp.broadcast_to(blk_mx[None, :], (8, HEADS))

    @pl.when(i == 0)
    def _():
        mx_ref[...] = blk_mx

    @pl.when(i > 0)
    def _():
        mx_ref[...] = jnp.maximum(mx_ref[...], blk_mx)


def _dense_stage(xp, lin_weight, att_src, att_dst):
    wt = lin_weight.T
    grid = NT // ROW_BLK
    h, asrc, adst, gb = pl.pallas_call(
        _dense_body,
        grid=(grid,),
        in_specs=[
            pl.BlockSpec((ROW_BLK, IN_CH), lambda i: (i, 0)),
            pl.BlockSpec((IN_CH, HEADS * OUT_CH), lambda i: (0, 0)),
            pl.BlockSpec((1, HEADS, OUT_CH), lambda i: (0, 0, 0)),
            pl.BlockSpec((1, HEADS, OUT_CH), lambda i: (0, 0, 0)),
        ],
        out_specs=[
            pl.BlockSpec((ROW_BLK, HEADS * OUT_CH), lambda i: (i, 0)),
            pl.BlockSpec((ROW_BLK, HEADS), lambda i: (i, 0)),
            pl.BlockSpec((ROW_BLK, HEADS), lambda i: (i, 0)),
            pl.BlockSpec((8, HEADS), lambda i: (0, 0)),
        ],
        out_shape=[
            jax.ShapeDtypeStruct((NT, HEADS * OUT_CH), jnp.float32),
            jax.ShapeDtypeStruct((NT, HEADS), jnp.float32),
            jax.ShapeDtypeStruct((NT, HEADS), jnp.float32),
            jax.ShapeDtypeStruct((8, HEADS), jnp.float32),
        ],
    )(xp, wt, att_src, att_dst)
    return h, asrc, adst, gb[0]


def _sc_a1_body(src_h, asrc_h, axr_h, tab, srcb, exb):
    c = lax.axis_index("c")
    s = lax.axis_index("s")
    pltpu.sync_copy(asrc_h.at[pl.ds(c * NT, NT)], tab)

    def _chunk(k, _):
        base = s * TPW + k * C
        pltpu.sync_copy(src_h.at[pl.ds(base, C)], srcb)

        def _grp(v, _):
            si = srcb[pl.ds(v * 16, 16)]
            exb[pl.ds(v * 16, 16)] = plsc.load_gather(tab, [si])
            return 0
        lax.fori_loop(0, C // 16, _grp, 0)
        pltpu.sync_copy(exb, axr_h.at[pl.ds(c * EP + base, C)])
        return 0
    lax.fori_loop(0, NK, _chunk, 0)


_sc_a1 = functools.partial(
    pl.kernel,
    _sc_a1_body,
    out_type=jax.ShapeDtypeStruct((QN,), jnp.float32),
    mesh=plsc.VectorSubcoreMesh(**_MESH),
    compiler_params=_CP,
    scratch_types=[
        pltpu.VMEM((NT,), jnp.float32),
        pltpu.VMEM((C,), jnp.int32),
        pltpu.VMEM((C,), jnp.float32),
    ],
)()


def _sc_a2_body(dst_h, axr_h, adst_h, gb_h, ex_h, den_h,
                denom_sp, tab, dstb, axb, exb, gbuf):
    c = lax.axis_index("c")
    s = lax.axis_index("s")
    zf = jnp.zeros((16,), jnp.float32)

    pltpu.sync_copy(gb_h, gbuf)
    gv = jnp.where(c == 0, gbuf[0], gbuf[1])
    pltpu.sync_copy(adst_h.at[pl.ds(c * NT, NT)], tab)

    def _zero(v, _):
        exb[pl.ds(v * 16, 16)] = zf
        return 0
    lax.fori_loop(0, C // 16, _zero, 0)
    for j in range(NPT // C):
        pltpu.sync_copy(exb, denom_sp.at[pl.ds(s * NPT + j * C, C)])
    _rm = NPT - (NPT // C) * C
    if _rm:
        pltpu.sync_copy(exb.at[pl.ds(0, _rm)],
                        denom_sp.at[pl.ds(s * NPT + (NPT // C) * C, _rm)])
    plsc.subcore_barrier()

    def _chunk(k, _):
        base = s * TPW + k * C
        pltpu.sync_copy(dst_h.at[pl.ds(base, C)], dstb)
        pltpu.sync_copy(axr_h.at[pl.ds(c * EP + base, C)], axb)

        def _grp(v, _):
            di = dstb[pl.ds(v * 16, 16)]
            al = axb[pl.ds(v * 16, 16)] + plsc.load_gather(tab, [di])
            al = jnp.where(al >= 0, al, al * NEG_SLOPE)
            exb[pl.ds(v * 16, 16)] = jnp.exp(al - gv)
            return 0
        lax.fori_loop(0, C // 16, _grp, 0)
        pltpu.sync_copy(exb, ex_h.at[pl.ds(c * EP + base, C)])
        pltpu.sync_copy(exb, denom_sp.at[dstb], add=True)
        return 0
    lax.fori_loop(0, NK, _chunk, 0)
    plsc.subcore_barrier()
    for j in range(NPT // C):
        pltpu.sync_copy(denom_sp.at[pl.ds(s * NPT + j * C, C)], exb)
        pltpu.sync_copy(exb, den_h.at[pl.ds(c * NT + s * NPT + j * C, C)])
    if _rm:
        _o = s * NPT + (NPT // C) * C
        pltpu.sync_copy(denom_sp.at[pl.ds(_o, _rm)], exb.at[pl.ds(0, _rm)])
        pltpu.sync_copy(exb.at[pl.ds(0, _rm)],
                        den_h.at[pl.ds(c * NT + _o, _rm)])


_sc_a2 = functools.partial(
    pl.kernel,
    _sc_a2_body,
    out_type=(jax.ShapeDtypeStruct((QN,), jnp.float32),
              jax.ShapeDtypeStruct((HN,), jnp.float32)),
    mesh=plsc.VectorSubcoreMesh(**_MESH),
    compiler_params=_CP,
    scratch_types=[
        pltpu.VMEM_SHARED((NT,), jnp.float32),
        pltpu.VMEM((NT,), jnp.float32),
        pltpu.VMEM((C,), jnp.int32),
        pltpu.VMEM((C,), jnp.float32),
        pltpu.VMEM((C,), jnp.float32),
        pltpu.VMEM((HEADS, 16), jnp.float32),
    ],
)()


def _sc_a3_body(src_h, dst_h, ex_h, den_h, h_h, msg_h,
                tab, srcb, dstb, exb, coefb, h_rows, msg, msg2, sem,
                sem2):
    c = lax.axis_index("c")
    s = lax.axis_index("s")

    pltpu.sync_copy(den_h.at[pl.ds(c * NT, NT)], tab)

    def _recip(v, _):
        d = tab[pl.ds(v * 16, 16)]
        tab[pl.ds(v * 16, 16)] = 1.0 / (d + 1e-16)
        return 0
    lax.fori_loop(0, NT // 16, _recip, 0)

    def _chunk(k, _):
        base = s * TPW + k * CA
        d1 = pltpu.async_copy(src_h.at[pl.ds(base, CA)], srcb, sem2)
        d2 = pltpu.async_copy(dst_h.at[pl.ds(base, CA)], dstb, sem2)
        d3 = pltpu.async_copy(ex_h.at[pl.ds(c * EP + base, CA)], exb, sem2)
        d1.wait()
        g = pltpu.async_copy(h_h.at[srcb], h_rows, sem)
        d2.wait()
        d3.wait()

        def _coef(v, _):
            di = dstb[pl.ds(v * 16, 16)]
            coefb[pl.ds(v * 16, 16)] = (exb[pl.ds(v * 16, 16)]
                                        * plsc.load_gather(tab, [di]))
            return 0
        lax.fori_loop(0, CA // 16, _coef, 0)
        g.wait()

        def _msg(g, _):
            cvec = coefb[pl.ds(g * 16, 16)]
            base_e = g * 16
            for j in range(16):
                e = base_e + j
                lo = jnp.where(c == 0, h_rows[e, pl.ds(0, 16)],
                               h_rows[e, pl.ds(32, 16)])
                hi = jnp.where(c == 0, h_rows[e, pl.ds(16, 16)],
                               h_rows[e, pl.ds(48, 16)])
                msg[pl.ds(e * 16, 16)] = lo * cvec[j]
                msg2[pl.ds(e * 16, 16)] = hi * cvec[j]
            return 0
        lax.fori_loop(0, CA // 16, _msg, 0)
        pltpu.sync_copy(msg, msg_h.at[pl.ds((c * EP + base) * 16, CA * 16)])
        pltpu.sync_copy(msg2,
                        msg_h.at[pl.ds((QN + c * EP + base) * 16, CA * 16)])
        return 0
    lax.fori_loop(0, NKA, _chunk, 0)


_sc_a3 = functools.partial(
    pl.kernel,
    _sc_a3_body,
    out_type=jax.ShapeDtypeStruct((2 * QN * 16,), jnp.float32),
    mesh=plsc.VectorSubcoreMesh(**_MESH),
    compiler_params=_CP,
    scratch_types=[
        pltpu.VMEM((NT,), jnp.float32),
        pltpu.VMEM((CA,), jnp.int32),
        pltpu.VMEM((CA,), jnp.int32),
        pltpu.VMEM((CA,), jnp.float32),
        pltpu.VMEM((CA,), jnp.float32),
        pltpu.VMEM((CA, 128), jnp.float32),
        pltpu.VMEM((CA * 16,), jnp.float32),
        pltpu.VMEM((CA * 16,), jnp.float32),
        pltpu.SemaphoreType.DMA,
        pltpu.SemaphoreType.DMA,
    ],
)()


def _sc_b_body(dst_h, msg_h, out_h, out_sp, dstd, idxf, msgf):
    c = lax.axis_index("c")
    s = lax.axis_index("s")
    zf = jnp.zeros((16,), jnp.float32)
    lanes = lax.iota(jnp.int32, 16)
    DW = D * 16
    NPW = NPT * 16
    NFD = NPW // DW
    NFR = NPW - NFD * DW

    def _zero(v, _):
        msgf[pl.ds(v * 16, 16)] = zf
        return 0

    def _zero_outsp():
        for j in range(NFD):
            pltpu.sync_copy(msgf, out_sp.at[pl.ds(s * NPW + j * DW, DW)])
        if NFR:
            pltpu.sync_copy(msgf.at[pl.ds(0, NFR)],
                            out_sp.at[pl.ds(s * NPW + NFD * DW, NFR)])

    def _flush(off):
        for j in range(NFD):
            pltpu.sync_copy(out_sp.at[pl.ds(s * NPW + j * DW, DW)], msgf)
            pltpu.sync_copy(msgf, out_h.at[pl.ds(off + j * DW, DW)])
        if NFR:
            pltpu.sync_copy(out_sp.at[pl.ds(s * NPW + NFD * DW, NFR)],
                            msgf.at[pl.ds(0, NFR)])
            pltpu.sync_copy(msgf.at[pl.ds(0, NFR)],
                            out_h.at[pl.ds(off + NFD * DW, NFR)])

    lax.fori_loop(0, DW // 16, _zero, 0)
    _zero_outsp()
    plsc.subcore_barrier()

    for f in range(2):
        def _drain(kk, _):
            base = s * TPW + kk * D
            pltpu.sync_copy(dst_h.at[pl.ds(base, D)], dstd)
            pltpu.sync_copy(msg_h.at[pl.ds((f * QN + c * EP + base) * 16,
                                           DW)], msgf)

            def _idx(v, _):
                dv = dstd[pl.ds(v * 16, 16)] * 16
                for j in range(16):
                    idxf[pl.ds((v * 16 + j) * 16, 16)] = dv[j] + lanes
                return 0
            lax.fori_loop(0, D // 16, _idx, 0)
            pltpu.sync_copy(msgf, out_sp.at[idxf], add=True)
            return 0
        lax.fori_loop(0, ND, _drain, 0)
        plsc.subcore_barrier()
        _flush((f * HN + c * NT + s * NPT) * 16)
        if f == 0:
            lax.fori_loop(0, DW // 16, _zero, 0)
            _zero_outsp()
            plsc.subcore_barrier()


_sc_b = functools.partial(
    pl.kernel,
    _sc_b_body,
    out_type=jax.ShapeDtypeStruct((2 * HN * 16,), jnp.float32),
    mesh=plsc.VectorSubcoreMesh(**_MESH),
    compiler_params=_CP,
    scratch_types=[
        pltpu.VMEM_SHARED((NT * 16,), jnp.float32),
        pltpu.VMEM((D,), jnp.int32),
        pltpu.VMEM((D * 16,), jnp.int32),
        pltpu.VMEM((D * 16,), jnp.float32),
    ],
)()


def kernel(x, edge_index, lin_weight, att_src, att_dst, bias):
    xp = jnp.pad(x, ((0, NT - N), (0, 0)))
    h, asrc, adst, gb = _dense_stage(xp, lin_weight, att_src, att_dst)

    h128 = jnp.pad(h, ((0, 0), (0, 128 - HEADS * OUT_CH)))
    asrc2 = asrc.T.reshape(HN)
    adst2 = adst.T.reshape(HN)
    gb2 = jnp.broadcast_to(gb[:, None], (HEADS, 16))

    loop = jnp.arange(N, dtype=jnp.int32)
    pad = jnp.full((EP - E - N,), TRASH, jnp.int32)
    src = jnp.concatenate([edge_index[0].astype(jnp.int32), loop, pad])
    dst = jnp.concatenate([edge_index[1].astype(jnp.int32), loop, pad])

    axr = _sc_a1(src, asrc2)
    ex, den = _sc_a2(dst, axr, adst2, gb2)
    msgq = _sc_a3(src, dst, ex, den, h128)
    oh = _sc_b(dst, msgq).reshape(2 * HN, 16)
    out = jnp.concatenate([oh[:N], oh[HN:HN + N],
                           oh[NT:NT + N], oh[HN + NT:HN + NT + N]],
                          axis=1) + bias
    return out


# speedup vs baseline: 31.8667x; 1.1122x over previous
import functools

import jax
import jax.numpy as jnp
from jax import lax
from jax.experimental import pallas as pl
from jax.experimental.pallas import tpu as pltpu
from jax.experimental.pallas import tpu_sc as plsc

N = 50000
E = 800000
IN_CH = 16
HEADS = 2
OUT_CH = 32
NEG_SLOPE = 0.2

ROW_BLK = 1024
NT = 50176
TRASH = N + 64
NSUB = 16
C = 320
NK = 167
TPW = C * NK
EP = NSUB * TPW
CA = 160
NKA = 334
D = 160
ND = TPW // D
NPT = NT // NSUB
NZD = 19
NZ = NPT - NZD * D
HN = HEADS * NT
QN = HEADS * EP

_MESH = dict(core_axis_name="c", subcore_axis_name="s")
_CP = pltpu.CompilerParams(needs_layout_passes=False)


def _dense_body(x_ref, wt_ref, asc_ref, adc_ref, h_ref, a_ref, b_ref, mx_ref):
    i = pl.program_id(0)
    xb = x_ref[...]
    h = lax.dot_general(xb, wt_ref[...], (((1,), (0,)), ((), ())),
                        preferred_element_type=jnp.float32)
    h_ref[...] = h
    hh = h.reshape(ROW_BLK, HEADS, OUT_CH)
    asrc = (hh * asc_ref[...]).sum(-1)
    adst = (hh * adc_ref[...]).sum(-1)
    a_ref[...] = asrc
    b_ref[...] = adst
    blk_mx = jnp.max(asrc, axis=0) + jnp.max(adst, axis=0)
    blk_mx = jnp.broadcast_to(blk_mx[None, :], (8, HEADS))

    @pl.when(i == 0)
    def _():
        mx_ref[...] = blk_mx

    @pl.when(i > 0)
    def _():
        mx_ref[...] = jnp.maximum(mx_ref[...], blk_mx)


def _dense_stage(xp, lin_weight, att_src, att_dst):
    wt = lin_weight.T
    grid = NT // ROW_BLK
    h, asrc, adst, gb = pl.pallas_call(
        _dense_body,
        grid=(grid,),
        in_specs=[
            pl.BlockSpec((ROW_BLK, IN_CH), lambda i: (i, 0)),
            pl.BlockSpec((IN_CH, HEADS * OUT_CH), lambda i: (0, 0)),
            pl.BlockSpec((1, HEADS, OUT_CH), lambda i: (0, 0, 0)),
            pl.BlockSpec((1, HEADS, OUT_CH), lambda i: (0, 0, 0)),
        ],
        out_specs=[
            pl.BlockSpec((ROW_BLK, HEADS * OUT_CH), lambda i: (i, 0)),
            pl.BlockSpec((ROW_BLK, HEADS), lambda i: (i, 0)),
            pl.BlockSpec((ROW_BLK, HEADS), lambda i: (i, 0)),
            pl.BlockSpec((8, HEADS), lambda i: (0, 0)),
        ],
        out_shape=[
            jax.ShapeDtypeStruct((NT, HEADS * OUT_CH), jnp.float32),
            jax.ShapeDtypeStruct((NT, HEADS), jnp.float32),
            jax.ShapeDtypeStruct((NT, HEADS), jnp.float32),
            jax.ShapeDtypeStruct((8, HEADS), jnp.float32),
        ],
    )(xp, wt, att_src, att_dst)
    return h, asrc, adst, gb[0]


def _sc_a1_body(src_h, asrc_h, axr_h, tab, srcb, exb):
    c = lax.axis_index("c")
    s = lax.axis_index("s")
    pltpu.sync_copy(asrc_h.at[pl.ds(c * NT, NT)], tab)

    def _chunk(k, _):
        base = s * TPW + k * C
        pltpu.sync_copy(src_h.at[pl.ds(base, C)], srcb)

        def _grp(v, _):
            si = srcb[pl.ds(v * 16, 16)]
            exb[pl.ds(v * 16, 16)] = plsc.load_gather(tab, [si])
            return 0
        lax.fori_loop(0, C // 16, _grp, 0)
        pltpu.sync_copy(exb, axr_h.at[pl.ds(c * EP + base, C)])
        return 0
    lax.fori_loop(0, NK, _chunk, 0)


_sc_a1 = functools.partial(
    pl.kernel,
    _sc_a1_body,
    out_type=jax.ShapeDtypeStruct((QN,), jnp.float32),
    mesh=plsc.VectorSubcoreMesh(**_MESH),
    compiler_params=_CP,
    scratch_types=[
        pltpu.VMEM((NT,), jnp.float32),
        pltpu.VMEM((C,), jnp.int32),
        pltpu.VMEM((C,), jnp.float32),
    ],
)()


def _sc_a2_body(dst_h, axr_h, adst_h, gb_h, ex_h, den_h,
                denom_sp, tab, dstb, axb, exb, gbuf, sem2):
    c = lax.axis_index("c")
    s = lax.axis_index("s")
    zf = jnp.zeros((16,), jnp.float32)

    pltpu.sync_copy(gb_h, gbuf)
    gv = jnp.where(c == 0, gbuf[0], gbuf[1])
    pltpu.sync_copy(adst_h.at[pl.ds(c * NT, NT)], tab)

    def _zero(v, _):
        exb[pl.ds(v * 16, 16)] = zf
        return 0
    lax.fori_loop(0, C // 16, _zero, 0)
    for j in range(NPT // C):
        pltpu.sync_copy(exb, denom_sp.at[pl.ds(s * NPT + j * C, C)])
    _rm = NPT - (NPT // C) * C
    if _rm:
        pltpu.sync_copy(exb.at[pl.ds(0, _rm)],
                        denom_sp.at[pl.ds(s * NPT + (NPT // C) * C, _rm)])
    plsc.subcore_barrier()

    def _chunk(k, _):
        base = s * TPW + k * C
        d1 = pltpu.async_copy(dst_h.at[pl.ds(base, C)], dstb, sem2)
        d2 = pltpu.async_copy(axr_h.at[pl.ds(c * EP + base, C)], axb, sem2)
        d1.wait()
        d2.wait()

        def _grp(v, _):
            di = dstb[pl.ds(v * 16, 16)]
            al = axb[pl.ds(v * 16, 16)] + plsc.load_gather(tab, [di])
            al = jnp.where(al >= 0, al, al * NEG_SLOPE)
            exb[pl.ds(v * 16, 16)] = jnp.exp(al - gv)
            return 0
        lax.fori_loop(0, C // 16, _grp, 0)
        pltpu.sync_copy(exb, ex_h.at[pl.ds(c * EP + base, C)])
        pltpu.sync_copy(exb, denom_sp.at[dstb], add=True)
        return 0
    lax.fori_loop(0, NK, _chunk, 0)
    plsc.subcore_barrier()
    for j in range(NPT // C):
        pltpu.sync_copy(denom_sp.at[pl.ds(s * NPT + j * C, C)], exb)
        pltpu.sync_copy(exb, den_h.at[pl.ds(c * NT + s * NPT + j * C, C)])
    if _rm:
        _o = s * NPT + (NPT // C) * C
        pltpu.sync_copy(denom_sp.at[pl.ds(_o, _rm)], exb.at[pl.ds(0, _rm)])
        pltpu.sync_copy(exb.at[pl.ds(0, _rm)],
                        den_h.at[pl.ds(c * NT + _o, _rm)])


_sc_a2 = functools.partial(
    pl.kernel,
    _sc_a2_body,
    out_type=(jax.ShapeDtypeStruct((QN,), jnp.float32),
              jax.ShapeDtypeStruct((HN,), jnp.float32)),
    mesh=plsc.VectorSubcoreMesh(**_MESH),
    compiler_params=_CP,
    scratch_types=[
        pltpu.VMEM_SHARED((NT,), jnp.float32),
        pltpu.VMEM((NT,), jnp.float32),
        pltpu.VMEM((C,), jnp.int32),
        pltpu.VMEM((C,), jnp.float32),
        pltpu.VMEM((C,), jnp.float32),
        pltpu.VMEM((HEADS, 16), jnp.float32),
        pltpu.SemaphoreType.DMA,
    ],
)()


def _sc_a3_body(src_h, dst_h, ex_h, den_h, h_h, msg_h,
                tab, srcb, dstb, exb, coefb, h_rows, msg, msg2, sem,
                sem2):
    c = lax.axis_index("c")
    s = lax.axis_index("s")

    pltpu.sync_copy(den_h.at[pl.ds(c * NT, NT)], tab)

    def _recip(v, _):
        d = tab[pl.ds(v * 16, 16)]
        tab[pl.ds(v * 16, 16)] = 1.0 / (d + 1e-16)
        return 0
    lax.fori_loop(0, NT // 16, _recip, 0)

    def _chunk(k, _):
        base = s * TPW + k * CA
        d1 = pltpu.async_copy(src_h.at[pl.ds(base, CA)], srcb, sem2)
        d2 = pltpu.async_copy(dst_h.at[pl.ds(base, CA)], dstb, sem2)
        d3 = pltpu.async_copy(ex_h.at[pl.ds(c * EP + base, CA)], exb, sem2)
        d1.wait()
        g = pltpu.async_copy(h_h.at[srcb], h_rows, sem)
        d2.wait()
        d3.wait()

        def _coef(v, _):
            di = dstb[pl.ds(v * 16, 16)]
            coefb[pl.ds(v * 16, 16)] = (exb[pl.ds(v * 16, 16)]
                                        * plsc.load_gather(tab, [di]))
            return 0
        lax.fori_loop(0, CA // 16, _coef, 0)
        g.wait()

        def _msg(g, _):
            cvec = coefb[pl.ds(g * 16, 16)]
            base_e = g * 16
            for j in range(16):
                e = base_e + j
                lo = jnp.where(c == 0, h_rows[e, pl.ds(0, 16)],
                               h_rows[e, pl.ds(32, 16)])
                hi = jnp.where(c == 0, h_rows[e, pl.ds(16, 16)],
                               h_rows[e, pl.ds(48, 16)])
                msg[pl.ds(e * 16, 16)] = lo * cvec[j]
                msg2[pl.ds(e * 16, 16)] = hi * cvec[j]
            return 0
        lax.fori_loop(0, CA // 16, _msg, 0)
        pltpu.sync_copy(msg, msg_h.at[pl.ds((c * EP + base) * 16, CA * 16)])
        pltpu.sync_copy(msg2,
                        msg_h.at[pl.ds((QN + c * EP + base) * 16, CA * 16)])
        return 0
    lax.fori_loop(0, NKA, _chunk, 0)


_sc_a3 = functools.partial(
    pl.kernel,
    _sc_a3_body,
    out_type=jax.ShapeDtypeStruct((2 * QN * 16,), jnp.float32),
    mesh=plsc.VectorSubcoreMesh(**_MESH),
    compiler_params=_CP,
    scratch_types=[
        pltpu.VMEM((NT,), jnp.float32),
        pltpu.VMEM((CA,), jnp.int32),
        pltpu.VMEM((CA,), jnp.int32),
        pltpu.VMEM((CA,), jnp.float32),
        pltpu.VMEM((CA,), jnp.float32),
        pltpu.VMEM((CA, 128), jnp.float32),
        pltpu.VMEM((CA * 16,), jnp.float32),
        pltpu.VMEM((CA * 16,), jnp.float32),
        pltpu.SemaphoreType.DMA,
        pltpu.SemaphoreType.DMA,
    ],
)()


def _sc_b_body(dst_h, msg_h, out_h, out_sp, dstd, idxf, msgf, semb):
    c = lax.axis_index("c")
    s = lax.axis_index("s")
    zf = jnp.zeros((16,), jnp.float32)
    lanes = lax.iota(jnp.int32, 16)
    DW = D * 16
    NPW = NPT * 16
    NFD = NPW // DW
    NFR = NPW - NFD * DW

    def _zero(v, _):
        msgf[pl.ds(v * 16, 16)] = zf
        return 0

    def _zero_outsp():
        for j in range(NFD):
            pltpu.sync_copy(msgf, out_sp.at[pl.ds(s * NPW + j * DW, DW)])
        if NFR:
            pltpu.sync_copy(msgf.at[pl.ds(0, NFR)],
                            out_sp.at[pl.ds(s * NPW + NFD * DW, NFR)])

    def _flush(off):
        for j in range(NFD):
            pltpu.sync_copy(out_sp.at[pl.ds(s * NPW + j * DW, DW)], msgf)
            pltpu.sync_copy(msgf, out_h.at[pl.ds(off + j * DW, DW)])
        if NFR:
            pltpu.sync_copy(out_sp.at[pl.ds(s * NPW + NFD * DW, NFR)],
                            msgf.at[pl.ds(0, NFR)])
            pltpu.sync_copy(msgf.at[pl.ds(0, NFR)],
                            out_h.at[pl.ds(off + NFD * DW, NFR)])

    lax.fori_loop(0, DW // 16, _zero, 0)
    _zero_outsp()
    plsc.subcore_barrier()

    for f in range(2):
        def _drain(kk, _):
            base = s * TPW + kk * D
            d1 = pltpu.async_copy(dst_h.at[pl.ds(base, D)], dstd, semb)
            d2 = pltpu.async_copy(
                msg_h.at[pl.ds((f * QN + c * EP + base) * 16, DW)], msgf,
                semb)
            d1.wait()
            d2.wait()

            def _idx(v, _):
                dv = dstd[pl.ds(v * 16, 16)] * 16
                for j in range(16):
                    idxf[pl.ds((v * 16 + j) * 16, 16)] = dv[j] + lanes
                return 0
            lax.fori_loop(0, D // 16, _idx, 0)
            pltpu.sync_copy(msgf, out_sp.at[idxf], add=True)
            return 0
        lax.fori_loop(0, ND, _drain, 0)
        plsc.subcore_barrier()
        _flush((f * HN + c * NT + s * NPT) * 16)
        if f == 0:
            lax.fori_loop(0, DW // 16, _zero, 0)
            _zero_outsp()
            plsc.subcore_barrier()


_sc_b = functools.partial(
    pl.kernel,
    _sc_b_body,
    out_type=jax.ShapeDtypeStruct((2 * HN * 16,), jnp.float32),
    mesh=plsc.VectorSubcoreMesh(**_MESH),
    compiler_params=_CP,
    scratch_types=[
        pltpu.VMEM_SHARED((NT * 16,), jnp.float32),
        pltpu.VMEM((D,), jnp.int32),
        pltpu.VMEM((D * 16,), jnp.int32),
        pltpu.VMEM((D * 16,), jnp.float32),
        pltpu.SemaphoreType.DMA,
    ],
)()


def kernel(x, edge_index, lin_weight, att_src, att_dst, bias):
    xp = jnp.pad(x, ((0, NT - N), (0, 0)))
    h, asrc, adst, gb = _dense_stage(xp, lin_weight, att_src, att_dst)

    h128 = jnp.pad(h, ((0, 0), (0, 128 - HEADS * OUT_CH)))
    asrc2 = asrc.T.reshape(HN)
    adst2 = adst.T.reshape(HN)
    gb2 = jnp.broadcast_to(gb[:, None], (HEADS, 16))

    loop = jnp.arange(N, dtype=jnp.int32)
    pad = jnp.full((EP - E - N,), TRASH, jnp.int32)
    src = jnp.concatenate([edge_index[0].astype(jnp.int32), loop, pad])
    dst = jnp.concatenate([edge_index[1].astype(jnp.int32), loop, pad])

    axr = _sc_a1(src, asrc2)
    ex, den = _sc_a2(dst, axr, adst2, gb2)
    msgq = _sc_a3(src, dst, ex, den, h128)
    oh = _sc_b(dst, msgq).reshape(2 * HN, 16)
    out = jnp.concatenate([oh[:N], oh[HN:HN + N],
                           oh[NT:NT + N], oh[HN + NT:HN + NT + N]],
                          axis=1) + bias
    return out
